# Initial kernel scaffold; baseline (speedup 1.0000x reference)
import jax
import jax.numpy as jnp
from jax.experimental import pallas as pl


def kernel(x, edge_index, edge_attr, W, b):
    raise NotImplementedError("write your pallas kernel here")


# speedup vs baseline: 16.8828x; 16.8828x over previous
import functools

import jax
import jax.numpy as jnp
from jax import lax
from jax.experimental import pallas as pl
from jax.experimental.pallas import tpu as pltpu
from jax.experimental.pallas import tpu_sc as plsc

NC = 2
NS = 16
NW = NC * NS
CHUNK = 128
BLK = 1000


def _mm_scale_body(dp_ref, x_ref, w_ref, g_ref):
    deg = dp_ref[0] + dp_ref[1] + 1.0
    h = jnp.dot(x_ref[...], w_ref[...], preferred_element_type=jnp.float32)
    g_ref[...] = lax.rsqrt(deg) * h


def _combine_body(s_ref, g_ref, dp_ref, b_ref, o_ref):
    deg = dp_ref[0] + dp_ref[1] + 1.0
    acc = s_ref[0] + s_ref[1] + g_ref[...]
    o_ref[...] = lax.rsqrt(deg) * acc + b_ref[...]


@functools.lru_cache(maxsize=None)
def _tc_calls(n, f):
    grid = (n // BLK,)
    row_spec = pl.BlockSpec((BLK, f), lambda i: (i, 0))
    dp_spec = pl.BlockSpec((NC, BLK, 1), lambda i: (0, i, 0))
    mm_scale = pl.pallas_call(
        _mm_scale_body,
        grid=grid,
        in_specs=[dp_spec, row_spec, pl.BlockSpec((f, f), lambda i: (0, 0))],
        out_specs=row_spec,
        out_shape=jax.ShapeDtypeStruct((n, f), jnp.float32),
    )
    combine = pl.pallas_call(
        _combine_body,
        grid=grid,
        in_specs=[pl.BlockSpec((NC, BLK, f), lambda i: (0, i, 0)),
                  row_spec, dp_spec,
                  pl.BlockSpec((1, f), lambda i: (0, 0))],
        out_specs=row_spec,
        out_shape=jax.ShapeDtypeStruct((n, f), jnp.float32),
    )
    return mm_scale, combine


@functools.lru_cache(maxsize=None)
def _sc_calls(n, f, cpt):
    out_rows = -(-n // (NS * 8)) * (NS * 8)
    out_pt = out_rows // NS
    acc_rows = out_rows
    init_pt = out_pt
    trash = n

    mesh = plsc.VectorSubcoreMesh(core_axis_name="c", subcore_axis_name="s")

    hrows = (out_rows + CHUNK) // CHUNK

    def _deg_kernel(colp, zeros_h, out_h, colv, hist, iotav, acc):
        cid = lax.axis_index("c")
        sid = lax.axis_index("s")
        wid = sid * NC + cid
        pltpu.sync_copy(zeros_h, hist)

        @pl.when(sid == 0)
        def _():
            pltpu.sync_copy(zeros_h, acc)

        pltpu.sync_copy(colp.at[wid], colv)
        for k in range(hrows // 16):
            iotav[pl.ds(k * 16, 16)] = lax.iota(jnp.int32, 16) + k * 16
        plsc.subcore_barrier()

        ones16 = jnp.ones((16,), jnp.float32)

        def body(j, carry):
            for k in range(CHUNK // 16):
                idx = colv[j, pl.ds(k * 16, 16)]
                plsc.addupdate_scatter(hist, [idx >> 7, idx & 127], ones16)
            return carry

        lax.fori_loop(0, cpt, body, 0)
        pltpu.sync_copy(hist, acc.at[iotav], add=True)
        plsc.subcore_barrier()

        @pl.when(sid < hrows // 8)
        def _():
            off = pl.multiple_of(sid * 8, 8)
            pltpu.sync_copy(acc.at[pl.ds(off, 8)],
                            out_h.at[cid, pl.ds(off, 8)])

    deg_call = pl.kernel(
        _deg_kernel,
        compiler_params=pltpu.CompilerParams(needs_layout_passes=False),
        out_type=jax.ShapeDtypeStruct((NC, hrows, CHUNK), jnp.float32),
        mesh=mesh,
        scratch_types=[
            pltpu.VMEM((cpt, CHUNK), jnp.int32),
            pltpu.VMEM((hrows, CHUNK), jnp.float32),
            pltpu.VMEM((hrows,), jnp.int32),
            pltpu.VMEM_SHARED((hrows, CHUNK), jnp.float32),
        ],
    )


    def _scat_kernel(g_h, rowp, colp, zeros_h, out_h, rowv, colv, buf, sem,
                     acc):
        cid = lax.axis_index("c")
        sid = lax.axis_index("s")
        wid = sid * NC + cid
        ioff = pl.multiple_of(sid * init_pt, 8)
        ooff = pl.multiple_of(sid * out_pt, 8)
        pltpu.sync_copy(zeros_h.at[pl.ds(ioff, init_pt)],
                        acc.at[pl.ds(ioff, init_pt)])
        pltpu.sync_copy(rowp.at[wid], rowv)
        pltpu.sync_copy(colp.at[wid], colv)
        plsc.subcore_barrier()

        def body(j, carry):
            pltpu.async_copy(g_h.at[rowv.at[j]], buf, sem).wait()
            pltpu.sync_copy(buf, acc.at[colv.at[j]], add=True)
            return carry

        lax.fori_loop(0, cpt, body, 0)
        plsc.subcore_barrier()
        pltpu.sync_copy(acc.at[pl.ds(ooff, out_pt)],
                        out_h.at[cid, pl.ds(ooff, out_pt)])

    scat_call = pl.kernel(
        _scat_kernel,
        out_type=jax.ShapeDtypeStruct((NC, out_rows, f), jnp.float32),
        mesh=mesh,
        scratch_types=[
            pltpu.VMEM((cpt, CHUNK), jnp.int32),
            pltpu.VMEM((cpt, CHUNK), jnp.int32),
            pltpu.VMEM((CHUNK, f), jnp.float32),
            pltpu.SemaphoreType.DMA,
            pltpu.VMEM_SHARED((acc_rows, f), jnp.float32),
        ],
    )

    return deg_call, scat_call, acc_rows, trash, hrows


def kernel(x, edge_index, edge_attr, W, b):
    del edge_attr
    n, f = x.shape
    e = edge_index.shape[1]
    cpt = -(-e // (NW * CHUNK))
    e_pad = NW * cpt * CHUNK
    pad = e_pad - e

    mm_scale, combine = _tc_calls(n, f)
    deg_call, scat_call, acc_rows, trash, hrows = _sc_calls(n, f, cpt)

    row = edge_index[0]
    col = edge_index[1]
    rowp2 = jnp.concatenate(
        [row, jnp.zeros((pad,), jnp.int32)]).reshape(NW, cpt, CHUNK)
    colp = jnp.concatenate(
        [col, jnp.full((pad,), trash, jnp.int32)]).reshape(NW, cpt, CHUNK)
    colp2 = colp

    zerosd = jnp.zeros((hrows, CHUNK), jnp.float32)
    zerosf = jnp.zeros((acc_rows, f), jnp.float32)

    degpart = deg_call(colp, zerosd)
    dp = degpart.reshape(NC, hrows * CHUNK)[:, :n].reshape(NC, n, 1)
    g = mm_scale(dp, x, W)
    s = scat_call(g, rowp2, colp2, zerosf)
    out = combine(s, g, dp, b.reshape(1, f))
    return out


# speedup vs baseline: 17.5152x; 1.0375x over previous
import functools

import jax
import jax.numpy as jnp
from jax import lax
from jax.experimental import pallas as pl
from jax.experimental.pallas import tpu as pltpu
from jax.experimental.pallas import tpu_sc as plsc

NC = 2
NS = 16
NW = NC * NS
CHUNK = 128
BLK = 1000


def _matmul_body(x_ref, w_ref, h_ref):
    h_ref[...] = jnp.dot(x_ref[...], w_ref[...],
                         preferred_element_type=jnp.float32)


def _scale_body(dp_ref, h_ref, g_ref):
    deg = dp_ref[0] + dp_ref[1] + 1.0
    g_ref[...] = lax.rsqrt(deg) * h_ref[...]


def _combine_body(s_ref, g_ref, dp_ref, b_ref, o_ref):
    deg = dp_ref[0] + dp_ref[1] + 1.0
    acc = s_ref[0] + s_ref[1] + g_ref[...]
    o_ref[...] = lax.rsqrt(deg) * acc + b_ref[...]


@functools.lru_cache(maxsize=None)
def _tc_calls(n, f):
    grid = (n // BLK,)
    row_spec = pl.BlockSpec((BLK, f), lambda i: (i, 0))
    dp_spec = pl.BlockSpec((NC, BLK, 1), lambda i: (0, i, 0))
    matmul = pl.pallas_call(
        _matmul_body,
        grid=grid,
        in_specs=[row_spec, pl.BlockSpec((f, f), lambda i: (0, 0))],
        out_specs=row_spec,
        out_shape=jax.ShapeDtypeStruct((n, f), jnp.float32),
    )
    scale = pl.pallas_call(
        _scale_body,
        grid=grid,
        in_specs=[dp_spec, row_spec],
        out_specs=row_spec,
        out_shape=jax.ShapeDtypeStruct((n, f), jnp.float32),
    )
    combine = pl.pallas_call(
        _combine_body,
        grid=grid,
        in_specs=[pl.BlockSpec((NC, BLK, f), lambda i: (0, i, 0)),
                  row_spec, dp_spec,
                  pl.BlockSpec((1, f), lambda i: (0, 0))],
        out_specs=row_spec,
        out_shape=jax.ShapeDtypeStruct((n, f), jnp.float32),
    )
    return matmul, scale, combine


@functools.lru_cache(maxsize=None)
def _sc_calls(n, f, cpt):
    out_rows = -(-n // (NS * 8)) * (NS * 8)
    out_pt = out_rows // NS
    acc_rows = out_rows
    init_pt = out_pt
    trash = n

    mesh = plsc.VectorSubcoreMesh(core_axis_name="c", subcore_axis_name="s")

    hrows = (out_rows + CHUNK) // CHUNK

    def _deg_kernel(colp, zeros_h, out_h, colv, hist, iotav, acc):
        cid = lax.axis_index("c")
        sid = lax.axis_index("s")
        wid = sid * NC + cid
        pltpu.sync_copy(zeros_h, hist)

        @pl.when(sid == 0)
        def _():
            pltpu.sync_copy(zeros_h, acc)

        pltpu.sync_copy(colp.at[wid], colv)
        for k in range(hrows // 16):
            iotav[pl.ds(k * 16, 16)] = lax.iota(jnp.int32, 16) + k * 16
        plsc.subcore_barrier()

        ones16 = jnp.ones((16,), jnp.float32)

        def body(j, carry):
            for k in range(CHUNK // 16):
                idx = colv[j, pl.ds(k * 16, 16)]
                plsc.addupdate_scatter(hist, [idx >> 7, idx & 127], ones16)
            return carry

        lax.fori_loop(0, cpt, body, 0)
        pltpu.sync_copy(hist, acc.at[iotav], add=True)
        plsc.subcore_barrier()

        @pl.when(sid < hrows // 8)
        def _():
            off = pl.multiple_of(sid * 8, 8)
            pltpu.sync_copy(acc.at[pl.ds(off, 8)],
                            out_h.at[cid, pl.ds(off, 8)])

    deg_call = pl.kernel(
        _deg_kernel,
        compiler_params=pltpu.CompilerParams(needs_layout_passes=False),
        out_type=jax.ShapeDtypeStruct((NC, hrows, CHUNK), jnp.float32),
        mesh=mesh,
        scratch_types=[
            pltpu.VMEM((cpt, CHUNK), jnp.int32),
            pltpu.VMEM((hrows, CHUNK), jnp.float32),
            pltpu.VMEM((hrows,), jnp.int32),
            pltpu.VMEM_SHARED((hrows, CHUNK), jnp.float32),
        ],
    )


    def _scat_kernel(g_h, rowp, colp, zeros_h, out_h, rowv, colv, buf, sem,
                     acc):
        cid = lax.axis_index("c")
        sid = lax.axis_index("s")
        wid = sid * NC + cid
        ioff = pl.multiple_of(sid * init_pt, 8)
        ooff = pl.multiple_of(sid * out_pt, 8)
        pltpu.sync_copy(zeros_h.at[pl.ds(ioff, init_pt)],
                        acc.at[pl.ds(ioff, init_pt)])
        pltpu.sync_copy(rowp.at[wid], rowv)
        pltpu.sync_copy(colp.at[wid], colv)
        plsc.subcore_barrier()

        def body(j, carry):
            pltpu.async_copy(g_h.at[rowv.at[j]], buf, sem).wait()
            pltpu.sync_copy(buf, acc.at[colv.at[j]], add=True)
            return carry

        lax.fori_loop(0, cpt, body, 0)
        plsc.subcore_barrier()
        pltpu.sync_copy(acc.at[pl.ds(ooff, out_pt)],
                        out_h.at[cid, pl.ds(ooff, out_pt)])

    scat_call = pl.kernel(
        _scat_kernel,
        out_type=jax.ShapeDtypeStruct((NC, out_rows, f), jnp.float32),
        mesh=mesh,
        scratch_types=[
            pltpu.VMEM((cpt, CHUNK), jnp.int32),
            pltpu.VMEM((cpt, CHUNK), jnp.int32),
            pltpu.VMEM((CHUNK, f), jnp.float32),
            pltpu.SemaphoreType.DMA,
            pltpu.VMEM_SHARED((acc_rows, f), jnp.float32),
        ],
    )

    return deg_call, scat_call, acc_rows, trash, hrows


def kernel(x, edge_index, edge_attr, W, b):
    del edge_attr
    n, f = x.shape
    e = edge_index.shape[1]
    cpt = -(-e // (NW * CHUNK))
    e_pad = NW * cpt * CHUNK
    pad = e_pad - e

    matmul, scale, combine = _tc_calls(n, f)
    deg_call, scat_call, acc_rows, trash, hrows = _sc_calls(n, f, cpt)

    row = edge_index[0]
    col = edge_index[1]
    rowp2 = jnp.concatenate(
        [row, jnp.zeros((pad,), jnp.int32)]).reshape(NW, cpt, CHUNK)
    colp = jnp.concatenate(
        [col, jnp.full((pad,), trash, jnp.int32)]).reshape(NW, cpt, CHUNK)
    colp2 = colp

    zerosd = jnp.zeros((hrows, CHUNK), jnp.float32)
    zerosf = jnp.zeros((acc_rows, f), jnp.float32)

    degpart = deg_call(colp, zerosd)
    dp = degpart.reshape(NC, hrows * CHUNK)[:, :n].reshape(NC, n, 1)
    h = matmul(x, W)
    g = scale(dp, h)
    s = scat_call(g, rowp2, colp2, zerosf)
    out = combine(s, g, dp, b.reshape(1, f))
    return out
